# Initial kernel scaffold; baseline (speedup 1.0000x reference)
import jax
import jax.numpy as jnp
from jax.experimental import pallas as pl


def kernel(input_ids, word_emb, pos_emb, msa_emb, ln_gamma, ln_beta):
    raise NotImplementedError("write your pallas kernel here")


# speedup vs baseline: 4.7444x; 4.7444x over previous
import functools

import jax
import jax.numpy as jnp
from jax.experimental import pallas as pl

B, A, S, H = 2, 32, 1024, 768
VOCAB = 26
VPAD = 32
SB = 256


def _embed_ln_kernel(ids_ref, w_ref, pos_ref, g_ref, b_ref, out_ref):
    ids = ids_ref[0, 0, :]
    idc = ids.reshape(SB, 1)
    oh = (idc == jax.lax.broadcasted_iota(jnp.int32, (SB, VPAD), 1))
    w = jnp.dot(oh.astype(jnp.float32), w_ref[...],
                preferred_element_type=jnp.float32)
    x = w + pos_ref[...]
    mean = jnp.mean(x, axis=-1, keepdims=True)
    xc = x - mean
    var = jnp.mean(xc * xc, axis=-1, keepdims=True)
    y = xc * jax.lax.rsqrt(var + 1e-12)
    y = y * g_ref[...] + b_ref[...]
    msk = (idc != 0).astype(jnp.float32)
    out_ref[...] = (y * msk)[None]


@jax.jit
def kernel(input_ids, word_emb, pos_emb, msa_emb, ln_gamma, ln_beta):
    del msa_emb
    BA = B * A
    ids3 = input_ids.reshape(BA, 1, S)
    wpad = jnp.zeros((VPAD, H), jnp.float32).at[:VOCAB].set(word_emb)
    pos_s = jax.lax.slice_in_dim(pos_emb, 2, 2 + S, axis=0)
    g2 = ln_gamma.reshape(1, H)
    b2 = ln_beta.reshape(1, H)

    out = pl.pallas_call(
        _embed_ln_kernel,
        grid=(S // SB, BA),
        in_specs=[
            pl.BlockSpec((1, 1, SB), lambda i_s, i_ba: (i_ba, 0, i_s)),
            pl.BlockSpec((VPAD, H), lambda i_s, i_ba: (0, 0)),
            pl.BlockSpec((SB, H), lambda i_s, i_ba: (i_s, 0)),
            pl.BlockSpec((1, H), lambda i_s, i_ba: (0, 0)),
            pl.BlockSpec((1, H), lambda i_s, i_ba: (0, 0)),
        ],
        out_specs=pl.BlockSpec((1, SB, H), lambda i_s, i_ba: (i_ba, i_s, 0)),
        out_shape=jax.ShapeDtypeStruct((BA, S, H), jnp.float32),
    )(ids3, wpad, pos_s, g2, b2)
    return out.reshape(B, A, S, H)
